# baseline (speedup 1.0000x reference)
import functools

import jax
import jax.numpy as jnp
from jax import lax
from jax.experimental import pallas as pl
from jax.experimental.pallas import tpu as pltpu
from jax.experimental.pallas import tpu_sc as plsc

KNN = 3
TN = 2048


def _dist_topk_body(x_ref, wt_ref, bias_ref, ytab_ref, idx_ref, dist_ref):
    bidx = pl.program_id(0)
    tile = pl.program_id(1)
    xb = x_ref[0]
    n_pts = xb.shape[1]
    n_rows = TN
    xsl = x_ref[0, :, pl.ds(tile * TN, TN)]
    xt = jnp.transpose(xsl)
    iota_i = lax.broadcasted_iota(jnp.int32, (n_rows, n_pts), 1)
    row_i = lax.broadcasted_iota(jnp.int32, (n_rows, n_pts), 0) + tile * n_rows
    diag_mask = iota_i == row_i
    iota_f = iota_i.astype(jnp.float32)
    ncol = jnp.sum(xb * xb, axis=0, keepdims=True)
    nrow = jnp.transpose(jnp.sum(xsl * xsl, axis=0, keepdims=True))
    dot = lax.dot_general(xt, xb, (((1,), (0,)), ((), ())),
                          preferred_element_type=jnp.float32)
    d2 = (ncol + nrow) - 2.0 * dot
    diag_d2 = jnp.sum(jnp.where(diag_mask, d2, 0.0), axis=1, keepdims=True)
    dist_ref[...] = jnp.where(diag_mask & (diag_d2 < 0.0),
                              jnp.float32(jnp.inf), d2)
    cand_v, cand_i = [], []
    for k in range(KNN + 1):
        d = dist_ref[...]
        m = jnp.min(d, axis=1, keepdims=True)
        sel = jnp.where(d == m, iota_f, jnp.float32(n_pts))
        amin_f = jnp.min(sel, axis=1, keepdims=True)
        cand_v.append(m)
        cand_i.append(amin_f)
        if k < KNN:
            dist_ref[...] = jnp.where(iota_f == amin_f, jnp.float32(jnp.inf), d)
    sq = [jnp.sqrt(v) for v in cand_v]
    order = list(range(KNN + 1))
    sv = list(sq)
    si = list(cand_i)
    for a in range(KNN):
        for bpos in range(KNN, a, -1):
            lo_first = ((sv[bpos] < sv[bpos - 1])
                        | ((sv[bpos] == sv[bpos - 1]) & (si[bpos] < si[bpos - 1])))
            va = jnp.where(lo_first, sv[bpos], sv[bpos - 1])
            vb = jnp.where(lo_first, sv[bpos - 1], sv[bpos])
            ia = jnp.where(lo_first, si[bpos], si[bpos - 1])
            ib = jnp.where(lo_first, si[bpos - 1], si[bpos])
            sv[bpos - 1], sv[bpos] = va, vb
            si[bpos - 1], si[bpos] = ia, ib
    cols = []
    for k in range(KNN):
        idx_i = si[k].astype(jnp.int32)
        cols.append(idx_i + (k * pl.num_programs(0) + bidx) * n_pts)
        yk = lax.dot_general(xt, wt_ref[k], (((1,), (0,)), ((), ())),
                             preferred_element_type=jnp.float32)
        if k == 0:
            yk = yk + bias_ref[...]
        ytab_ref[k, 0] = yk
    pad = jnp.zeros_like(cols[0])
    idx_ref[0] = jnp.concatenate(cols + [pad] * (8 - KNN), axis=1)


def _dist_topk(x, wt, bias2):
    B, C, N = x.shape
    O = wt.shape[2]
    return pl.pallas_call(
        _dist_topk_body,
        grid=(B, N // TN),
        in_specs=[
            pl.BlockSpec((1, C, N), lambda b, i: (b, 0, 0)),
            pl.BlockSpec((KNN, C, O), lambda b, i: (0, 0, 0)),
            pl.BlockSpec((1, O), lambda b, i: (0, 0)),
        ],
        out_specs=[
            pl.BlockSpec((KNN, 1, TN, O), lambda b, i: (0, b, i, 0)),
            pl.BlockSpec((1, TN, 8), lambda b, i: (b, i, 0)),
        ],
        out_shape=[
            jax.ShapeDtypeStruct((KNN, B, N, O), jnp.float32),
            jax.ShapeDtypeStruct((B, N, 8), jnp.int32),
        ],
        scratch_shapes=[pltpu.VMEM((TN, N), jnp.float32)],
    )(x, wt, bias2)


def _make_sc_gather(total_rows, n_chunks, feat):
    mesh = plsc.VectorSubcoreMesh(core_axis_name="c", subcore_axis_name="s")
    n_workers = 32

    @functools.partial(
        pl.kernel,
        mesh=mesh,
        compiler_params=pltpu.CompilerParams(use_tc_tiling_on_sc=False),
        out_type=jax.ShapeDtypeStruct((n_workers, n_chunks, 128, feat),
                                      jnp.float32),
        scratch_types=[
            pltpu.VMEM((KNN, n_chunks, 128), jnp.int32),
            pltpu.VMEM((n_chunks, 128, feat), jnp.float32),
            pltpu.VMEM((n_chunks, 128, feat), jnp.float32),
            pltpu.VMEM((n_chunks, 128, feat), jnp.float32),
            pltpu.SemaphoreType.DMA,
        ],
    )
    def sc_gather(ytab_hbm, idx_hbm, out_hbm, idxv, r0, r1, r2, sem):
        wid = lax.axis_index("s") * 2 + lax.axis_index("c")
        pltpu.sync_copy(idx_hbm.at[:, wid], idxv)
        bufs = (r0, r1, r2)
        copies = []
        for k in range(KNN):
            for j in range(n_chunks):
                copies.append(
                    pltpu.async_copy(ytab_hbm.at[idxv.at[k, j]],
                                     bufs[k].at[j], sem))
        for c in copies:
            c.wait()

        def body(i, carry):
            j = i // 128
            r = i % 128
            for c4 in range(feat // 16):
                s = pl.ds(c4 * 16, 16)
                r0[j, r, s] = r0[j, r, s] + r1[j, r, s] + r2[j, r, s]
            return carry

        lax.fori_loop(0, n_chunks * 128, body, 0)
        pltpu.sync_copy(r0, out_hbm.at[wid])

    return sc_gather


def kernel(x, W, b):
    B, C, N = x.shape
    O = W.shape[0]
    wt = jnp.transpose(W, (2, 1, 0))
    bias2 = b.reshape(1, O)

    ytab, idxs = _dist_topk(x, wt, bias2)
    ytab_flat = ytab.reshape(KNN * B * N, O)

    n_workers = 32
    n_chunks = (B * N) // (n_workers * 128)
    idx_all = jnp.transpose(idxs[:, :, :KNN], (2, 0, 1)).reshape(
        KNN, n_workers, n_chunks, 128)

    out_rows = _make_sc_gather(B * N, n_chunks, O)(ytab_flat, idx_all)
    return out_rows.reshape(B, N, O).transpose(0, 2, 1)


# speedup vs baseline: 1.0200x; 1.0200x over previous
import functools

import jax
import jax.numpy as jnp
from jax import lax
from jax.experimental import pallas as pl
from jax.experimental.pallas import tpu as pltpu
from jax.experimental.pallas import tpu_sc as plsc

KNN = 3
TN = 2048


def _dist_topk_body(x_ref, wt_ref, bias_ref, ytab_ref, idx_ref, dist_ref):
    bidx = pl.program_id(0)
    tile = pl.program_id(1)
    xb = x_ref[0]
    n_pts = xb.shape[1]
    n_rows = TN
    xsl = x_ref[0, :, pl.ds(tile * TN, TN)]
    xt = jnp.transpose(xsl)
    iota_i = lax.broadcasted_iota(jnp.int32, (n_rows, n_pts), 1)
    row_i = lax.broadcasted_iota(jnp.int32, (n_rows, n_pts), 0) + tile * n_rows
    diag_mask = iota_i == row_i
    iota_f = iota_i.astype(jnp.float32)
    ncol = jnp.sum(xb * xb, axis=0, keepdims=True)
    nrow = jnp.transpose(jnp.sum(xsl * xsl, axis=0, keepdims=True))
    dot = lax.dot_general(xt, xb, (((1,), (0,)), ((), ())),
                          preferred_element_type=jnp.float32)
    d2 = (ncol + nrow) - 2.0 * dot
    diag_d2 = jnp.sum(jnp.where(diag_mask, d2, 0.0), axis=1, keepdims=True)
    dist_ref[...] = jnp.where(diag_mask & (diag_d2 < 0.0),
                              jnp.float32(jnp.inf), d2)
    cand_v, cand_i = [], []
    for k in range(KNN + 1):
        d = dist_ref[...]
        m = jnp.min(d, axis=1, keepdims=True)
        sel = jnp.where(d == m, iota_f, jnp.float32(n_pts))
        amin_f = jnp.min(sel, axis=1, keepdims=True)
        cand_v.append(m)
        cand_i.append(amin_f)
        if k < KNN:
            dist_ref[...] = jnp.where(iota_f == amin_f, jnp.float32(jnp.inf), d)
    sq = [jnp.sqrt(v) for v in cand_v]
    order = list(range(KNN + 1))
    sv = list(sq)
    si = list(cand_i)
    for a in range(KNN):
        for bpos in range(KNN, a, -1):
            lo_first = ((sv[bpos] < sv[bpos - 1])
                        | ((sv[bpos] == sv[bpos - 1]) & (si[bpos] < si[bpos - 1])))
            va = jnp.where(lo_first, sv[bpos], sv[bpos - 1])
            vb = jnp.where(lo_first, sv[bpos - 1], sv[bpos])
            ia = jnp.where(lo_first, si[bpos], si[bpos - 1])
            ib = jnp.where(lo_first, si[bpos - 1], si[bpos])
            sv[bpos - 1], sv[bpos] = va, vb
            si[bpos - 1], si[bpos] = ia, ib
    cols = []
    for k in range(KNN):
        idx_i = si[k].astype(jnp.int32)
        cols.append(idx_i + (k * pl.num_programs(0) + bidx) * n_pts)
        yk = lax.dot_general(xt, wt_ref[k], (((1,), (0,)), ((), ())),
                             preferred_element_type=jnp.float32)
        if k == 0:
            yk = yk + bias_ref[...]
        ytab_ref[k, 0] = jnp.concatenate([yk, jnp.zeros_like(yk)], axis=1)
    pad = jnp.zeros_like(cols[0])
    idx_ref[0] = jnp.concatenate(cols + [pad] * (8 - KNN), axis=1)


def _dist_topk(x, wt, bias2):
    B, C, N = x.shape
    O = wt.shape[2]
    return pl.pallas_call(
        _dist_topk_body,
        grid=(B, N // TN),
        in_specs=[
            pl.BlockSpec((1, C, N), lambda b, i: (b, 0, 0)),
            pl.BlockSpec((KNN, C, O), lambda b, i: (0, 0, 0)),
            pl.BlockSpec((1, O), lambda b, i: (0, 0)),
        ],
        out_specs=[
            pl.BlockSpec((KNN, 1, TN, 2 * O), lambda b, i: (0, b, i, 0)),
            pl.BlockSpec((1, TN, 8), lambda b, i: (b, i, 0)),
        ],
        out_shape=[
            jax.ShapeDtypeStruct((KNN, B, N, 2 * O), jnp.float32),
            jax.ShapeDtypeStruct((B, N, 8), jnp.int32),
        ],
        scratch_shapes=[pltpu.VMEM((TN, N), jnp.float32)],
    )(x, wt, bias2)


def _make_sc_gather(n_chunks, feat):
    mesh = plsc.VectorSubcoreMesh(core_axis_name="c", subcore_axis_name="s")
    n_workers = 32
    half = n_chunks // 2

    @functools.partial(
        pl.kernel,
        mesh=mesh,
        compiler_params=pltpu.CompilerParams(use_tc_tiling_on_sc=False),
        out_type=jax.ShapeDtypeStruct((n_workers, 2, half * 128, feat),
                                      jnp.float32),
        scratch_types=[
            pltpu.VMEM((KNN, n_chunks, 128), jnp.int32),
            pltpu.VMEM((half, 128, 2 * feat), jnp.float32),
            pltpu.VMEM((half, 128, 2 * feat), jnp.float32),
            pltpu.VMEM((half, 128, 2 * feat), jnp.float32),
            pltpu.VMEM((half * 128, feat), jnp.float32),
            pltpu.SemaphoreType.DMA,
        ],
    )
    def sc_gather(ytab_hbm, idx_hbm, out_hbm, idxv, g0, g1, g2, ob, sem):
        wid = lax.axis_index("s") * 2 + lax.axis_index("c")
        pltpu.sync_copy(idx_hbm.at[:, wid], idxv)
        bufs = (g0, g1, g2)
        for h in range(2):
            copies = []
            for k in range(KNN):
                for j in range(half):
                    copies.append(
                        pltpu.async_copy(ytab_hbm.at[idxv.at[k, h * half + j]],
                                         bufs[k].at[j], sem))
            for c in copies:
                c.wait()

            def body(i, carry):
                j = i // 128
                r = i % 128
                for c4 in range(feat // 16):
                    s = pl.ds(c4 * 16, 16)
                    ob[i, s] = g0[j, r, s] + g1[j, r, s] + g2[j, r, s]
                return carry

            lax.fori_loop(0, half * 128, body, 0)
            pltpu.sync_copy(ob, out_hbm.at[wid, h])

    return sc_gather


def kernel(x, W, b):
    B, C, N = x.shape
    O = W.shape[0]
    wt = jnp.transpose(W, (2, 1, 0))
    bias2 = b.reshape(1, O)

    ytab, idxs = _dist_topk(x, wt, bias2)
    ytab_flat = ytab.reshape(KNN * B * N, 2 * O)

    n_workers = 32
    n_chunks = (B * N) // (n_workers * 128)
    idx_all = jnp.transpose(idxs[:, :, :KNN], (2, 0, 1)).reshape(
        KNN, n_workers, n_chunks, 128)

    out_rows = _make_sc_gather(n_chunks, O)(ytab_flat, idx_all)
    return out_rows.reshape(B, N, O).transpose(0, 2, 1)


# speedup vs baseline: 1.0495x; 1.0290x over previous
import functools

import jax
import jax.numpy as jnp
from jax import lax
from jax.experimental import pallas as pl
from jax.experimental.pallas import tpu as pltpu
from jax.experimental.pallas import tpu_sc as plsc

KNN = 3
TN = 2048


def _dist_topk_body(x_ref, wt_ref, bias_ref, ytab_ref, idx_ref, dist_ref):
    bidx = pl.program_id(0)
    tile = pl.program_id(1)
    xb = x_ref[0]
    n_pts = xb.shape[1]
    n_rows = TN
    xsl = x_ref[0, :, pl.ds(tile * TN, TN)]
    xt = jnp.transpose(xsl)
    iota_i = lax.broadcasted_iota(jnp.int32, (n_rows, n_pts), 1)
    row_i = lax.broadcasted_iota(jnp.int32, (n_rows, n_pts), 0) + tile * n_rows
    diag_mask = iota_i == row_i
    iota_f = iota_i.astype(jnp.float32)
    ncol = jnp.sum(xb * xb, axis=0, keepdims=True)
    nrow = jnp.transpose(jnp.sum(xsl * xsl, axis=0, keepdims=True))
    dot = lax.dot_general(xt, xb, (((1,), (0,)), ((), ())),
                          preferred_element_type=jnp.float32)
    d2 = (ncol + nrow) - 2.0 * dot
    diag_d2 = jnp.sum(jnp.where(diag_mask, d2, 0.0), axis=1, keepdims=True)
    dist_ref[...] = jnp.where(diag_mask & (diag_d2 < 0.0),
                              jnp.float32(jnp.inf), d2)
    cand_v, cand_i = [], []
    for k in range(KNN + 1):
        d = dist_ref[...]
        m = jnp.min(d, axis=1, keepdims=True)
        sel = jnp.where(d == m, iota_f, jnp.float32(n_pts))
        amin_f = jnp.min(sel, axis=1, keepdims=True)
        cand_v.append(m)
        cand_i.append(amin_f)
        if k < KNN:
            dist_ref[...] = jnp.where(iota_f == amin_f, jnp.float32(jnp.inf), d)
    sq = [jnp.sqrt(v) for v in cand_v]
    order = list(range(KNN + 1))
    sv = list(sq)
    si = list(cand_i)
    for a in range(KNN):
        for bpos in range(KNN, a, -1):
            lo_first = ((sv[bpos] < sv[bpos - 1])
                        | ((sv[bpos] == sv[bpos - 1]) & (si[bpos] < si[bpos - 1])))
            va = jnp.where(lo_first, sv[bpos], sv[bpos - 1])
            vb = jnp.where(lo_first, sv[bpos - 1], sv[bpos])
            ia = jnp.where(lo_first, si[bpos], si[bpos - 1])
            ib = jnp.where(lo_first, si[bpos - 1], si[bpos])
            sv[bpos - 1], sv[bpos] = va, vb
            si[bpos - 1], si[bpos] = ia, ib
    cols = []
    for k in range(KNN):
        idx_i = si[k].astype(jnp.int32)
        cols.append(idx_i + (k * pl.num_programs(0) + bidx) * n_pts)
        yk = lax.dot_general(xt, wt_ref[k], (((1,), (0,)), ((), ())),
                             preferred_element_type=jnp.float32)
        if k == 0:
            yk = yk + bias_ref[...]
        ytab_ref[k, 0] = jnp.concatenate([yk, jnp.zeros_like(yk)], axis=1)
    pad = jnp.zeros_like(cols[0])
    idx_ref[0] = jnp.concatenate(cols + [pad] * (8 - KNN), axis=1)


def _dist_topk(x, wt, bias2):
    B, C, N = x.shape
    O = wt.shape[2]
    return pl.pallas_call(
        _dist_topk_body,
        grid=(B, N // TN),
        in_specs=[
            pl.BlockSpec((1, C, N), lambda b, i: (b, 0, 0)),
            pl.BlockSpec((KNN, C, O), lambda b, i: (0, 0, 0)),
            pl.BlockSpec((1, O), lambda b, i: (0, 0)),
        ],
        out_specs=[
            pl.BlockSpec((KNN, 1, TN, 2 * O), lambda b, i: (0, b, i, 0)),
            pl.BlockSpec((1, TN, 8), lambda b, i: (b, i, 0)),
        ],
        out_shape=[
            jax.ShapeDtypeStruct((KNN, B, N, 2 * O), jnp.float32),
            jax.ShapeDtypeStruct((B, N, 8), jnp.int32),
        ],
        scratch_shapes=[pltpu.VMEM((TN, N), jnp.float32)],
    )(x, wt, bias2)


def _make_sc_gather(n_chunks, feat):
    mesh = plsc.VectorSubcoreMesh(core_axis_name="c", subcore_axis_name="s")
    n_workers = 32

    @functools.partial(
        pl.kernel,
        mesh=mesh,
        compiler_params=pltpu.CompilerParams(use_tc_tiling_on_sc=False),
        out_type=jax.ShapeDtypeStruct((n_workers, n_chunks, 128, feat),
                                      jnp.float32),
        scratch_types=[
            pltpu.VMEM((KNN, n_chunks, 128), jnp.int32),
            pltpu.VMEM((2, KNN, 128, 2 * feat), jnp.float32),
            pltpu.VMEM((128, feat), jnp.float32),
            pltpu.SemaphoreType.DMA,
            pltpu.SemaphoreType.DMA,
        ],
    )
    def sc_gather(ytab_hbm, idx_hbm, out_hbm, idxv, gbuf, ob, sem0, sem1):
        wid = lax.axis_index("s") * 2 + lax.axis_index("c")
        pltpu.sync_copy(idx_hbm.at[:, wid], idxv)
        sems = (sem0, sem1)

        def fire(q):
            bs = q % 2
            return [
                pltpu.async_copy(ytab_hbm.at[idxv.at[k, q]],
                                 gbuf.at[bs, k], sems[bs])
                for k in range(KNN)
            ]

        pending = fire(0)
        for q in range(n_chunks):
            nxt = fire(q + 1) if q + 1 < n_chunks else None
            for c in pending:
                c.wait()
            bs = q % 2

            def body(r, carry):
                for c4 in range(feat // 16):
                    s = pl.ds(c4 * 16, 16)
                    ob[r, s] = (gbuf[bs, 0, r, s] + gbuf[bs, 1, r, s]
                                + gbuf[bs, 2, r, s])
                return carry

            lax.fori_loop(0, 128, body, 0)
            pltpu.sync_copy(ob, out_hbm.at[wid, q])
            pending = nxt

    return sc_gather


def kernel(x, W, b):
    B, C, N = x.shape
    O = W.shape[0]
    wt = jnp.transpose(W, (2, 1, 0))
    bias2 = b.reshape(1, O)

    ytab, idxs = _dist_topk(x, wt, bias2)
    ytab_flat = ytab.reshape(KNN * B * N, 2 * O)

    n_workers = 32
    n_chunks = (B * N) // (n_workers * 128)
    idx_all = jnp.transpose(idxs[:, :, :KNN], (2, 0, 1)).reshape(
        KNN, n_workers, n_chunks, 128)

    out_rows = _make_sc_gather(n_chunks, O)(ytab_flat, idx_all)
    return out_rows.reshape(B, N, O).transpose(0, 2, 1)


# speedup vs baseline: 1.0928x; 1.0413x over previous
import functools

import jax
import jax.numpy as jnp
from jax import lax
from jax.experimental import pallas as pl
from jax.experimental.pallas import tpu as pltpu
from jax.experimental.pallas import tpu_sc as plsc

KNN = 3
TN = 2048


def _dist_topk_body(x_ref, wt_ref, bias_ref, ytab_ref, idx_ref, dist_ref):
    bidx = pl.program_id(0)
    tile = pl.program_id(1)
    xb = x_ref[0]
    n_pts = xb.shape[1]
    n_rows = TN
    xsl = x_ref[0, :, pl.ds(tile * TN, TN)]
    xt = jnp.transpose(xsl)
    iota_i = lax.broadcasted_iota(jnp.int32, (n_rows, n_pts), 1)
    row_i = lax.broadcasted_iota(jnp.int32, (n_rows, n_pts), 0) + tile * n_rows
    diag_mask = iota_i == row_i
    iota_f = iota_i.astype(jnp.float32)
    ncol = jnp.sum(xb * xb, axis=0, keepdims=True)
    nrow = jnp.transpose(jnp.sum(xsl * xsl, axis=0, keepdims=True))
    dot = lax.dot_general(xt, xb, (((1,), (0,)), ((), ())),
                          preferred_element_type=jnp.float32)
    d2 = (ncol + nrow) - 2.0 * dot
    diag_d2 = jnp.sum(jnp.where(diag_mask, d2, 0.0), axis=1, keepdims=True)
    dist_ref[...] = jnp.where(diag_mask & (diag_d2 < 0.0),
                              jnp.float32(jnp.inf), d2)
    cand_v, cand_i = [], []
    for k in range(KNN + 1):
        d = dist_ref[...]
        m = jnp.min(d, axis=1, keepdims=True)
        sel = jnp.where(d == m, iota_f, jnp.float32(n_pts))
        amin_f = jnp.min(sel, axis=1, keepdims=True)
        cand_v.append(m)
        cand_i.append(amin_f)
        if k < KNN:
            dist_ref[...] = jnp.where(iota_f == amin_f, jnp.float32(jnp.inf), d)
    sq = [jnp.sqrt(v) for v in cand_v]
    order = list(range(KNN + 1))
    sv = list(sq)
    si = list(cand_i)
    for a in range(KNN):
        for bpos in range(KNN, a, -1):
            lo_first = ((sv[bpos] < sv[bpos - 1])
                        | ((sv[bpos] == sv[bpos - 1]) & (si[bpos] < si[bpos - 1])))
            va = jnp.where(lo_first, sv[bpos], sv[bpos - 1])
            vb = jnp.where(lo_first, sv[bpos - 1], sv[bpos])
            ia = jnp.where(lo_first, si[bpos], si[bpos - 1])
            ib = jnp.where(lo_first, si[bpos - 1], si[bpos])
            sv[bpos - 1], sv[bpos] = va, vb
            si[bpos - 1], si[bpos] = ia, ib
    cols = []
    for k in range(KNN):
        idx_i = si[k].astype(jnp.int32)
        cols.append(idx_i + (k * pl.num_programs(0) + bidx) * n_pts)
        yk = lax.dot_general(xt, wt_ref[k], (((1,), (0,)), ((), ())),
                             preferred_element_type=jnp.float32)
        if k == 0:
            yk = yk + bias_ref[...]
        ytab_ref[k, 0] = jnp.concatenate([yk, jnp.zeros_like(yk)], axis=1)
    for k in range(KNN):
        idx_ref[k] = cols[k].reshape(n_rows // 512, 4, 128)


def _dist_topk(x, wt, bias2):
    B, C, N = x.shape
    O = wt.shape[2]
    return pl.pallas_call(
        _dist_topk_body,
        grid=(B, N // TN),
        in_specs=[
            pl.BlockSpec((1, C, N), lambda b, i: (b, 0, 0)),
            pl.BlockSpec((KNN, C, O), lambda b, i: (0, 0, 0)),
            pl.BlockSpec((1, O), lambda b, i: (0, 0)),
        ],
        out_specs=[
            pl.BlockSpec((KNN, 1, TN, 2 * O), lambda b, i: (0, b, i, 0)),
            pl.BlockSpec((KNN, TN // 512, 4, 128), lambda b, i: (0, b, i, 0)),
        ],
        out_shape=[
            jax.ShapeDtypeStruct((KNN, B, N, 2 * O), jnp.float32),
            jax.ShapeDtypeStruct((KNN, (B * N) // 512, 4, 128), jnp.int32),
        ],
        scratch_shapes=[pltpu.VMEM((TN, N), jnp.float32)],
    )(x, wt, bias2)


def _make_sc_gather(n_chunks, feat):
    mesh = plsc.VectorSubcoreMesh(core_axis_name="c", subcore_axis_name="s")
    n_workers = 32

    @functools.partial(
        pl.kernel,
        mesh=mesh,
        compiler_params=pltpu.CompilerParams(use_tc_tiling_on_sc=False),
        out_type=jax.ShapeDtypeStruct((n_workers, n_chunks, 128, feat),
                                      jnp.float32),
        scratch_types=[
            pltpu.VMEM((KNN, n_chunks, 128), jnp.int32),
            pltpu.VMEM((2, KNN, 128, 2 * feat), jnp.float32),
            pltpu.VMEM((128, feat), jnp.float32),
            pltpu.SemaphoreType.DMA,
            pltpu.SemaphoreType.DMA,
        ],
    )
    def sc_gather(ytab_hbm, idx_hbm, out_hbm, idxv, gbuf, ob, sem0, sem1):
        wid = lax.axis_index("s") * 2 + lax.axis_index("c")
        pltpu.sync_copy(idx_hbm.at[:, wid], idxv)
        sems = (sem0, sem1)

        def fire(q):
            bs = q % 2
            return [
                pltpu.async_copy(ytab_hbm.at[idxv.at[k, q]],
                                 gbuf.at[bs, k], sems[bs])
                for k in range(KNN)
            ]

        pending = fire(0)
        for q in range(n_chunks):
            nxt = fire(q + 1) if q + 1 < n_chunks else None
            for c in pending:
                c.wait()
            bs = q % 2

            def body(r, carry):
                for c4 in range(feat // 16):
                    s = pl.ds(c4 * 16, 16)
                    ob[r, s] = (gbuf[bs, 0, r, s] + gbuf[bs, 1, r, s]
                                + gbuf[bs, 2, r, s])
                return carry

            lax.fori_loop(0, 128, body, 0)
            pltpu.sync_copy(ob, out_hbm.at[wid, q])
            pending = nxt

    return sc_gather


def kernel(x, W, b):
    B, C, N = x.shape
    O = W.shape[0]
    wt = jnp.transpose(W, (2, 1, 0))
    bias2 = b.reshape(1, O)

    ytab, idxs = _dist_topk(x, wt, bias2)
    ytab_flat = ytab.reshape(KNN * B * N, 2 * O)

    n_workers = 32
    n_chunks = (B * N) // (n_workers * 128)
    out_rows = _make_sc_gather(n_chunks, O)(ytab_flat, idxs)
    return out_rows.reshape(B, N, O).transpose(0, 2, 1)
